# baseline (speedup 1.0000x reference)
import jax
import jax.numpy as jnp
from jax import lax
from jax.experimental import pallas as pl
from jax.experimental.pallas import tpu as pltpu
from jax.experimental.pallas import tpu_sc as plsc

_ROWS = 16384
_COLS = 64
_NC = 2
_NS = 16
_NW = _NC * _NS
_RPW = _ROWS // _NW


def _body(x_hbm, o_hbm, buf):
    wid = lax.axis_index("s") * _NC + lax.axis_index("c")
    base = wid * _RPW
    pltpu.sync_copy(x_hbm.at[pl.ds(base, _RPW)], buf)
    pltpu.sync_copy(buf, o_hbm.at[pl.ds(base, _RPW)])


def kernel(x, memoryData, memoryTarget):
    mesh = plsc.VectorSubcoreMesh(core_axis_name="c", subcore_axis_name="s")
    import functools
    k = functools.partial(
        pl.kernel,
        mesh=mesh,
        out_type=jax.ShapeDtypeStruct((_ROWS, _COLS), jnp.float32),
        scratch_types=[pltpu.VMEM((_RPW, _COLS), jnp.float32)],
    )(_body)
    return k(x)


# speedup vs baseline: 1.7867x; 1.7867x over previous
import jax
import jax.numpy as jnp
from jax.experimental import pallas as pl
from jax.experimental.pallas import tpu as pltpu

_ROWS = 16384
_COLS = 64
_NBUF = 8
_BLK = _ROWS // _NBUF


def _copy_body(x_ref, o_ref, bufs, in_sems, out_sems):
    for j in range(_NBUF):
        pltpu.make_async_copy(
            x_ref.at[pl.ds(j * _BLK, _BLK)], bufs.at[j], in_sems.at[j]
        ).start()
    for j in range(_NBUF):
        pltpu.make_async_copy(
            x_ref.at[pl.ds(j * _BLK, _BLK)], bufs.at[j], in_sems.at[j]
        ).wait()
        pltpu.make_async_copy(
            bufs.at[j], o_ref.at[pl.ds(j * _BLK, _BLK)], out_sems.at[j]
        ).start()
    for j in range(_NBUF):
        pltpu.make_async_copy(
            bufs.at[j], o_ref.at[pl.ds(j * _BLK, _BLK)], out_sems.at[j]
        ).wait()


def kernel(x, memoryData, memoryTarget):
    return pl.pallas_call(
        _copy_body,
        out_shape=jax.ShapeDtypeStruct(x.shape, x.dtype),
        in_specs=[pl.BlockSpec(memory_space=pltpu.HBM)],
        out_specs=pl.BlockSpec(memory_space=pltpu.HBM),
        scratch_shapes=[
            pltpu.VMEM((_NBUF, _BLK, _COLS), jnp.float32),
            pltpu.SemaphoreType.DMA((_NBUF,)),
            pltpu.SemaphoreType.DMA((_NBUF,)),
        ],
    )(x)


# speedup vs baseline: 1.7927x; 1.0034x over previous
import jax
import jax.numpy as jnp
from jax.experimental import pallas as pl
from jax.experimental.pallas import tpu as pltpu

_ROWS = 16384
_COLS = 64
_NBUF = 8
_BLK = _ROWS // _NBUF


def _copy_body(x_ref, o_ref, bufs, in_sems, out_sems):
    for j in range(_NBUF):
        pltpu.make_async_copy(
            x_ref.at[pl.ds(j * _BLK, _BLK)], bufs.at[j], in_sems.at[j]
        ).start()
    for j in range(_NBUF):
        pltpu.make_async_copy(
            x_ref.at[pl.ds(j * _BLK, _BLK)], bufs.at[j], in_sems.at[j]
        ).wait()
        pltpu.make_async_copy(
            bufs.at[j], o_ref.at[pl.ds(j * _BLK, _BLK)], out_sems.at[j]
        ).start()
    for j in range(_NBUF):
        pltpu.make_async_copy(
            bufs.at[j], o_ref.at[pl.ds(j * _BLK, _BLK)], out_sems.at[j]
        ).wait()


def kernel(x, memoryData, memoryTarget):
    return pl.pallas_call(
        _copy_body,
        out_shape=jax.ShapeDtypeStruct(x.shape, x.dtype),
        in_specs=[pl.BlockSpec(memory_space=pl.ANY)],
        out_specs=pl.BlockSpec(memory_space=pl.ANY),
        scratch_shapes=[
            pltpu.VMEM((_NBUF, _BLK, _COLS), jnp.float32),
            pltpu.SemaphoreType.DMA((_NBUF,)),
            pltpu.SemaphoreType.DMA((_NBUF,)),
        ],
    )(x)
